# baseline (speedup 1.0000x reference)
import functools

import jax
import jax.numpy as jnp
from jax import lax
from jax.experimental import pallas as pl
from jax.experimental.pallas import tpu as pltpu
from jax.experimental.pallas import tpu_sc as plsc

SPACE_SIZE = 100000
OUT_DIM = 16
FIELD = 26
BATCH = 16384

_C = OUT_DIM * FIELD
_info = plsc.get_sparse_core_info()
_NC, _NS = _info.num_cores, _info.num_subcores
_NW = _NC * _NS
_C_PER_W = _C // _NW
_OCHUNK = 1024
_NBUF = 8


@functools.partial(
    pl.kernel,
    mesh=plsc.VectorSubcoreMesh(core_axis_name="c", subcore_axis_name="s"),
    out_type=jax.ShapeDtypeStruct((_C, BATCH), jnp.float32),
    scratch_types=[
        pltpu.VMEM((BATCH,), jnp.int32),
        pltpu.VMEM((SPACE_SIZE,), jnp.float32),
        pltpu.VMEM((_NBUF, _OCHUNK), jnp.float32),
        pltpu.SemaphoreType.DMA,
    ],
    compiler_params=pltpu.CompilerParams(needs_layout_passes=False),
)
def _lookup(table_hbm, idx_hbm, out_hbm, idx_v, col_v, ob, wsem):
    wid = lax.axis_index("s") * _NC + lax.axis_index("c")
    pltpu.sync_copy(idx_hbm, idx_v)

    nchunk = BATCH // _OCHUNK
    for ci in range(_C_PER_W):
        col = wid * _C_PER_W + ci
        pltpu.sync_copy(table_hbm.at[col], col_v)
        for ch in range(nchunk):
            gchunk = ci * nchunk + ch
            buf = gchunk % _NBUF

            def grp(g, carry, _ch=ch, _buf=buf):
                i16 = idx_v[pl.ds(_ch * _OCHUNK + g * 16, 16)]
                ob[_buf, pl.ds(g * 16, 16)] = plsc.load_gather(col_v, [i16])
                return carry

            if gchunk >= _NBUF:
                pltpu.make_async_copy(
                    ob.at[buf], out_hbm.at[col, pl.ds(0, _OCHUNK)], wsem
                ).wait()
            lax.fori_loop(0, _OCHUNK // 16, grp, 0)
            pltpu.async_copy(
                ob.at[buf], out_hbm.at[col, pl.ds(ch * _OCHUNK, _OCHUNK)], wsem
            )
    for _ in range(_NBUF):
        pltpu.make_async_copy(
            ob.at[0], out_hbm.at[0, pl.ds(0, _OCHUNK)], wsem
        ).wait()


def kernel(x, ker):
    kt = jnp.transpose(ker, (2, 1, 0)).reshape(_C, SPACE_SIZE)
    out = _lookup(kt, x.astype(jnp.int32))
    return jnp.transpose(out.reshape(FIELD, OUT_DIM, BATCH), (2, 1, 0))[..., None]


# speedup vs baseline: 1.0128x; 1.0128x over previous
import functools

import jax
import jax.numpy as jnp
from jax import lax
from jax.experimental import pallas as pl
from jax.experimental.pallas import tpu as pltpu
from jax.experimental.pallas import tpu_sc as plsc

SPACE_SIZE = 100000
OUT_DIM = 16
FIELD = 26
BATCH = 16384

_C = OUT_DIM * FIELD
_info = plsc.get_sparse_core_info()
_NC, _NS = _info.num_cores, _info.num_subcores
_NW = _NC * _NS
_C_PER_W = _C // _NW
_OCHUNK = 2048
_NBUF = 2
_UNROLL = 1


@functools.partial(
    pl.kernel,
    mesh=plsc.VectorSubcoreMesh(core_axis_name="c", subcore_axis_name="s"),
    out_type=jax.ShapeDtypeStruct((_C, BATCH), jnp.float32),
    scratch_types=[
        pltpu.VMEM((BATCH,), jnp.int32),
        pltpu.VMEM((SPACE_SIZE,), jnp.float32),
        pltpu.VMEM((_NBUF, _OCHUNK), jnp.float32),
        pltpu.SemaphoreType.DMA,
    ],
    compiler_params=pltpu.CompilerParams(needs_layout_passes=False),
)
def _lookup(table_hbm, idx_hbm, out_hbm, idx_v, col_v, ob, wsem):
    wid = lax.axis_index("s") * _NC + lax.axis_index("c")
    pltpu.sync_copy(idx_hbm, idx_v)

    nchunk = BATCH // _OCHUNK
    for ci in range(_C_PER_W):
        col = wid * _C_PER_W + ci
        pltpu.sync_copy(table_hbm.at[col], col_v)
        for ch in range(nchunk):
            gchunk = ci * nchunk + ch
            buf = gchunk % _NBUF

            def grp(g, carry, _ch=ch, _buf=buf):
                for u in range(_UNROLL):
                    off = g * (16 * _UNROLL) + u * 16
                    i16 = idx_v[pl.ds(_ch * _OCHUNK + off, 16)]
                    ob[_buf, pl.ds(off, 16)] = plsc.load_gather(col_v, [i16])
                return carry

            if gchunk >= _NBUF:
                pltpu.make_async_copy(
                    ob.at[buf], out_hbm.at[col, pl.ds(0, _OCHUNK)], wsem
                ).wait()
            lax.fori_loop(0, _OCHUNK // (16 * _UNROLL), grp, 0)
            pltpu.async_copy(
                ob.at[buf], out_hbm.at[col, pl.ds(ch * _OCHUNK, _OCHUNK)], wsem
            )
    for _ in range(_NBUF):
        pltpu.make_async_copy(
            ob.at[0], out_hbm.at[0, pl.ds(0, _OCHUNK)], wsem
        ).wait()


def kernel(x, ker):
    kt = jnp.transpose(ker, (2, 1, 0)).reshape(_C, SPACE_SIZE)
    out = _lookup(kt, x.astype(jnp.int32))
    return jnp.transpose(out.reshape(FIELD, OUT_DIM, BATCH), (2, 1, 0))[..., None]


# speedup vs baseline: 1.1507x; 1.1362x over previous
import functools

import jax
import jax.numpy as jnp
from jax import lax
from jax.experimental import pallas as pl
from jax.experimental.pallas import tpu as pltpu
from jax.experimental.pallas import tpu_sc as plsc

SPACE_SIZE = 100000
OUT_DIM = 16
FIELD = 26
BATCH = 16384

_C = OUT_DIM * FIELD
_info = plsc.get_sparse_core_info()
_NC, _NS = _info.num_cores, _info.num_subcores
_NW = _NC * _NS
_C_PER_W = _C // _NW
_OCHUNK = 2048
_NBUF = 2


@functools.partial(
    pl.kernel,
    mesh=plsc.VectorSubcoreMesh(core_axis_name="c", subcore_axis_name="s"),
    out_type=jax.ShapeDtypeStruct((_C * 8, BATCH // 8), jnp.float32),
    scratch_types=[
        pltpu.VMEM((BATCH,), jnp.int32),
        pltpu.VMEM((SPACE_SIZE,), jnp.float32),
        pltpu.VMEM((_NBUF, 8, _OCHUNK // 8), jnp.float32),
        pltpu.SemaphoreType.DMA,
    ],
    compiler_params=pltpu.CompilerParams(needs_layout_passes=False),
)
def _lookup(table_hbm, idx_hbm, out_hbm, idx_v, col_v, ob, wsem):
    wid = lax.axis_index("s") * _NC + lax.axis_index("c")
    pltpu.sync_copy(idx_hbm, idx_v)

    nchunk = BATCH // _OCHUNK
    kpc = _OCHUNK // 1024
    for ci in range(_C_PER_W):
        col = wid * _C_PER_W + ci
        c_out = lax.rem(col, OUT_DIM) * FIELD + col // OUT_DIM
        pltpu.sync_copy(table_hbm.at[col], col_v)
        for ch in range(nchunk):
            gchunk = ci * nchunk + ch
            buf = gchunk % _NBUF

            def grp(g, carry, _ch=ch, _buf=buf):
                pj = g // (kpc * 8)
                pk = (g // 8) % kpc
                pl_ = g % 8
                b_off = (pk * 8 + pj) * 128 + pl_ * 16
                i16 = idx_v[pl.ds(_ch * _OCHUNK + b_off, 16)]
                ob[_buf, pj, pl.ds(pk * 128 + pl_ * 16, 16)] = (
                    plsc.load_gather(col_v, [i16])
                )
                return carry

            if gchunk >= _NBUF:
                pltpu.make_async_copy(
                    ob.at[buf],
                    out_hbm.at[pl.ds(0, 8), pl.ds(0, _OCHUNK // 8)],
                    wsem,
                ).wait()
            lax.fori_loop(0, _OCHUNK // 16, grp, 0)
            pltpu.async_copy(
                ob.at[buf],
                out_hbm.at[
                    pl.ds(c_out * 8, 8),
                    pl.ds(ch * (_OCHUNK // 8), _OCHUNK // 8),
                ],
                wsem,
            )
    for _ in range(_NBUF):
        pltpu.make_async_copy(
            ob.at[0], out_hbm.at[pl.ds(0, 8), pl.ds(0, _OCHUNK // 8)], wsem
        ).wait()


def kernel(x, ker):
    kt = jnp.transpose(ker, (2, 1, 0)).reshape(_C, SPACE_SIZE)
    out = _lookup(kt, x.astype(jnp.int32))
    out = out.reshape(_C, 8, BATCH // 1024, 128)
    out = jnp.transpose(out, (2, 1, 3, 0)).reshape(BATCH, _C)
    return out.reshape(BATCH, OUT_DIM, FIELD)[..., None]


# speedup vs baseline: 1.1527x; 1.0017x over previous
import functools

import jax
import jax.numpy as jnp
from jax import lax
from jax.experimental import pallas as pl
from jax.experimental.pallas import tpu as pltpu
from jax.experimental.pallas import tpu_sc as plsc

SPACE_SIZE = 100000
OUT_DIM = 16
FIELD = 26
BATCH = 16384

_C = OUT_DIM * FIELD
_info = plsc.get_sparse_core_info()
_NC, _NS = _info.num_cores, _info.num_subcores
_NW = _NC * _NS
_C_PER_W = _C // _NW
_OCHUNK = 2048
_NBUF = 3


@functools.partial(
    pl.kernel,
    mesh=plsc.VectorSubcoreMesh(core_axis_name="c", subcore_axis_name="s"),
    out_type=jax.ShapeDtypeStruct((_C * 8, BATCH // 8), jnp.float32),
    scratch_types=[
        pltpu.VMEM((BATCH,), jnp.int32),
        pltpu.VMEM((SPACE_SIZE,), jnp.float32),
        pltpu.VMEM((_NBUF, 8, _OCHUNK // 8), jnp.float32),
        pltpu.SemaphoreType.DMA,
    ],
    compiler_params=pltpu.CompilerParams(needs_layout_passes=False),
)
def _lookup(table_hbm, idx_hbm, out_hbm, idx_v, col_v, ob, wsem):
    wid = lax.axis_index("s") * _NC + lax.axis_index("c")
    pltpu.sync_copy(idx_hbm, idx_v)

    nchunk = BATCH // _OCHUNK
    kpc = _OCHUNK // 1024
    for ci in range(_C_PER_W):
        col = wid * _C_PER_W + ci
        c_out = lax.rem(col, OUT_DIM) * FIELD + col // OUT_DIM
        pltpu.sync_copy(table_hbm.at[col], col_v)
        for ch in range(nchunk):
            gchunk = ci * nchunk + ch
            buf = gchunk % _NBUF

            def grp(g, carry, _ch=ch, _buf=buf):
                pj = g // (kpc * 8)
                pk = (g // 8) % kpc
                pl_ = g % 8
                b_off = (pk * 8 + pj) * 128 + pl_ * 16
                i16 = idx_v[pl.ds(_ch * _OCHUNK + b_off, 16)]
                ob[_buf, pj, pl.ds(pk * 128 + pl_ * 16, 16)] = (
                    plsc.load_gather(col_v, [i16])
                )
                return carry

            if gchunk >= _NBUF:
                pltpu.make_async_copy(
                    ob.at[buf],
                    out_hbm.at[pl.ds(0, 8), pl.ds(0, _OCHUNK // 8)],
                    wsem,
                ).wait()
            lax.fori_loop(0, _OCHUNK // 16, grp, 0)
            pltpu.async_copy(
                ob.at[buf],
                out_hbm.at[
                    pl.ds(c_out * 8, 8),
                    pl.ds(ch * (_OCHUNK // 8), _OCHUNK // 8),
                ],
                wsem,
            )
    for _ in range(_NBUF):
        pltpu.make_async_copy(
            ob.at[0], out_hbm.at[pl.ds(0, 8), pl.ds(0, _OCHUNK // 8)], wsem
        ).wait()


def kernel(x, ker):
    kt = jnp.transpose(ker, (2, 1, 0)).reshape(_C, SPACE_SIZE)
    out = _lookup(kt, x.astype(jnp.int32))
    out = out.reshape(_C, 8, BATCH // 1024, 128)
    out = jnp.transpose(out, (2, 1, 3, 0)).reshape(BATCH, _C)
    return out.reshape(BATCH, OUT_DIM, FIELD)[..., None]


# speedup vs baseline: 2.0250x; 1.7568x over previous
import functools

import jax
import jax.numpy as jnp
from jax import lax
from jax.experimental import pallas as pl
from jax.experimental.pallas import tpu as pltpu
from jax.experimental.pallas import tpu_sc as plsc

SPACE_SIZE = 100000
OUT_DIM = 16
FIELD = 26
BATCH = 16384

_C = OUT_DIM * FIELD
_info = plsc.get_sparse_core_info()
_NC, _NS = _info.num_cores, _info.num_subcores
_NW = _NC * _NS
_C_PER_W = _C // _NW
_OCHUNK = 2048
_NBUF = 3


@functools.partial(
    pl.kernel,
    mesh=plsc.VectorSubcoreMesh(core_axis_name="c", subcore_axis_name="s"),
    out_type=jax.ShapeDtypeStruct((_C * 8, BATCH // 8), jnp.float32),
    scratch_types=[
        pltpu.VMEM((BATCH,), jnp.int32),
        pltpu.VMEM((SPACE_SIZE,), jnp.float32),
        pltpu.VMEM((_NBUF, 8, _OCHUNK // 8), jnp.float32),
        pltpu.SemaphoreType.DMA,
    ],
    compiler_params=pltpu.CompilerParams(needs_layout_passes=False),
)
def _lookup(table_hbm, idx_hbm, out_hbm, idx_v, col_v, ob, wsem):
    wid = lax.axis_index("s") * _NC + lax.axis_index("c")
    pltpu.sync_copy(idx_hbm, idx_v)

    nchunk = BATCH // _OCHUNK
    kpc = _OCHUNK // 1024
    for ci in range(_C_PER_W):
        col = wid * _C_PER_W + ci
        c_out = lax.rem(col, OUT_DIM) * FIELD + col // OUT_DIM
        pltpu.sync_copy(table_hbm.at[col], col_v)
        for ch in range(nchunk):
            gchunk = ci * nchunk + ch
            buf = gchunk % _NBUF

            def grp(g, _ch=ch, _buf=buf):
                pj = g // (kpc * 8)
                pk = (g // 8) % kpc
                pl_ = g % 8
                b_off = (pk * 8 + pj) * 128 + pl_ * 16
                i16 = idx_v[pl.ds(_ch * _OCHUNK + b_off, 16)]
                ob[_buf, pj, pl.ds(pk * 128 + pl_ * 16, 16)] = (
                    plsc.load_gather(col_v, [i16])
                )

            if gchunk >= _NBUF:
                pltpu.make_async_copy(
                    ob.at[buf],
                    out_hbm.at[pl.ds(0, 8), pl.ds(0, _OCHUNK // 8)],
                    wsem,
                ).wait()
            plsc.parallel_loop(0, _OCHUNK // 16, 1, unroll=4)(grp)
            pltpu.async_copy(
                ob.at[buf],
                out_hbm.at[
                    pl.ds(c_out * 8, 8),
                    pl.ds(ch * (_OCHUNK // 8), _OCHUNK // 8),
                ],
                wsem,
            )
    for _ in range(_NBUF):
        pltpu.make_async_copy(
            ob.at[0], out_hbm.at[pl.ds(0, 8), pl.ds(0, _OCHUNK // 8)], wsem
        ).wait()


def kernel(x, ker):
    kt = jnp.transpose(ker, (2, 1, 0)).reshape(_C, SPACE_SIZE)
    out = _lookup(kt, x.astype(jnp.int32))
    out = out.reshape(_C, 8, BATCH // 1024, 128)
    out = jnp.transpose(out, (2, 1, 3, 0)).reshape(BATCH, _C)
    return out.reshape(BATCH, OUT_DIM, FIELD)[..., None]


# speedup vs baseline: 2.0432x; 1.0090x over previous
import functools

import jax
import jax.numpy as jnp
from jax import lax
from jax.experimental import pallas as pl
from jax.experimental.pallas import tpu as pltpu
from jax.experimental.pallas import tpu_sc as plsc

SPACE_SIZE = 100000
OUT_DIM = 16
FIELD = 26
BATCH = 16384

_C = OUT_DIM * FIELD
_info = plsc.get_sparse_core_info()
_NC, _NS = _info.num_cores, _info.num_subcores
_NW = _NC * _NS
_C_PER_W = _C // _NW
_OCHUNK = 2048
_NBUF = 3


@functools.partial(
    pl.kernel,
    mesh=plsc.VectorSubcoreMesh(core_axis_name="c", subcore_axis_name="s"),
    out_type=jax.ShapeDtypeStruct((_C * 8, BATCH // 8), jnp.float32),
    scratch_types=[
        pltpu.VMEM((BATCH,), jnp.int32),
        pltpu.VMEM((SPACE_SIZE,), jnp.float32),
        pltpu.VMEM((_NBUF, 8, _OCHUNK // 8), jnp.float32),
        pltpu.SemaphoreType.DMA,
    ],
    compiler_params=pltpu.CompilerParams(needs_layout_passes=False),
)
def _lookup(table_hbm, idx_hbm, out_hbm, idx_v, col_v, ob, wsem):
    wid = lax.axis_index("s") * _NC + lax.axis_index("c")
    pltpu.sync_copy(idx_hbm, idx_v)

    nchunk = BATCH // _OCHUNK
    kpc = _OCHUNK // 1024
    for ci in range(_C_PER_W):
        col = wid * _C_PER_W + ci
        c_out = lax.rem(col, OUT_DIM) * FIELD + col // OUT_DIM
        pltpu.sync_copy(table_hbm.at[col], col_v)
        for ch in range(nchunk):
            gchunk = ci * nchunk + ch
            buf = gchunk % _NBUF

            def grp(g, _ch=ch, _buf=buf):
                pj = g // (kpc * 8)
                pk = (g // 8) % kpc
                pl_ = g % 8
                b_off = (pk * 8 + pj) * 128 + pl_ * 16
                i16 = idx_v[pl.ds(_ch * _OCHUNK + b_off, 16)]
                ob[_buf, pj, pl.ds(pk * 128 + pl_ * 16, 16)] = (
                    plsc.load_gather(col_v, [i16])
                )

            if gchunk >= _NBUF:
                pltpu.make_async_copy(
                    ob.at[buf],
                    out_hbm.at[pl.ds(0, 8), pl.ds(0, _OCHUNK // 8)],
                    wsem,
                ).wait()
            plsc.parallel_loop(0, _OCHUNK // 16, 1, unroll=8)(grp)
            pltpu.async_copy(
                ob.at[buf],
                out_hbm.at[
                    pl.ds(c_out * 8, 8),
                    pl.ds(ch * (_OCHUNK // 8), _OCHUNK // 8),
                ],
                wsem,
            )
    for _ in range(_NBUF):
        pltpu.make_async_copy(
            ob.at[0], out_hbm.at[pl.ds(0, 8), pl.ds(0, _OCHUNK // 8)], wsem
        ).wait()


def kernel(x, ker):
    kt = jnp.transpose(ker, (2, 1, 0)).reshape(_C, SPACE_SIZE)
    out = _lookup(kt, x.astype(jnp.int32))
    out = out.reshape(_C, 8, BATCH // 1024, 128)
    out = jnp.transpose(out, (2, 1, 3, 0)).reshape(BATCH, _C)
    return out.reshape(BATCH, OUT_DIM, FIELD)[..., None]
